# baseline (speedup 1.0000x reference)
import functools

import jax
import jax.numpy as jnp
from jax import lax
from jax.experimental import pallas as pl
from jax.experimental.pallas import tpu as pltpu
from jax.experimental.pallas import tpu_sc as plsc

BATCH = 16384
EMB = 64
NC = 2
NS = 16
LANES = 16
NW = NC * NS
BPW = BATCH // NW
CHUNK = 128
NCH = BPW // CHUNK
NROW = 1000000
NJ = (NROW + 127) // 128
NJL = NROW // 128
JPW = 246
TAIL0 = NJL * 128
TAILN = NROW - TAIL0
LCAP = BATCH + LANES
ITMP = 4096
RING = 16
DEPTH = 2
NBK = 544


def _scalar(x):
    return x if getattr(x, "ndim", 0) == 0 else x[0]


def _extract_body(user_h, item_h, uet_h, iet_h, tu_h, ti_h,
                  uo_h, io_h,
                  itmp, ul, il, ublk, iblk, tu_v, ti_v,
                  stg, drn, cnt_s, blksem, outsem):
    wid = lax.axis_index("s") * NC + lax.axis_index("c")
    jlo = wid * JPW
    jhi = jnp.minimum(jlo + JPW, NJ)
    jhb = jnp.minimum(jhi, NJL)
    jcnt = jhb - jlo
    iota = lax.iota(jnp.int32, LANES)

    pltpu.sync_copy(tu_h, tu_v)
    pltpu.sync_copy(ti_h, ti_v)

    def build(idx_h, lst):
        def piece(s, n):
            pltpu.sync_copy(idx_h.at[pl.ds(s * ITMP, ITMP)], itmp)

            def sup(c, n):
                for q in range(8):
                    rv = itmp[pl.ds(c * 128 + q * LANES, LANES)]
                    jv = lax.shift_right_logical(rv, 7)
                    m = (jv >= jlo) & (jv < jhi)
                    ent = (lax.shift_left(jv - jlo, 21)
                           | lax.shift_left(rv & 127, 14)
                           | (s * ITMP + c * 128 + q * LANES + iota))
                    plsc.store_compressed(lst.at[pl.ds(n, LANES)], ent,
                                          mask=m)
                    n = n + _scalar(plsc.all_reduce_population_count(m))
                return n

            return lax.fori_loop(0, ITMP // 128, sup, n)

        n = jnp.int32(0)
        for s in range(BATCH // ITMP):
            n = piece(s, n)
        return n

    nu = build(user_h, ul)
    ni = build(item_h, il)

    cnt_s[0] = 0

    def fire_row(pos):
        c = cnt_s[0]
        slot = c & (RING - 1)

        @pl.when(c >= RING)
        def _():
            pltpu.make_async_copy(uo_h.at[pl.ds(0, EMB)], drn, outsem).wait()

        cnt_s[0] = c + 1
        return slot

    def scan(lst, n, jrel, extract):
        nv = lax.shift_right_logical(n + 127, 7)

        def sup(c, _):
            evs, ms = [], []
            for q in range(8):
                base_e = c * 128 + q * LANES
                ev = lst[pl.ds(base_e, LANES)]
                m = ((lax.shift_right_logical(ev, 21) == jrel)
                     & ((base_e + iota) < n))
                evs.append(ev)
                ms.append(m)
            big = ms[0]
            for q in range(1, 8):
                big = big | ms[q]

            @pl.when(jnp.any(big))
            def _():
                for q in range(8):
                    base_e = c * 128 + q * LANES

                    @pl.when(jnp.any(ms[q]))
                    def _():
                        def w_cond(m_):
                            return jnp.any(m_)

                        def w_body(m_):
                            lane = _scalar(plsc.all_reduce_ffs(m_))
                            e = plsc.load_gather(
                                lst, [jnp.full((LANES,), base_e + lane,
                                               jnp.int32)])[0]
                            extract(e & 0x3FFF,
                                    lax.shift_right_logical(e, 14) & 127)
                            return m_ & (iota != lane)

                        lax.while_loop(w_cond, w_body, ms[q])

            return _

        lax.fori_loop(0, nv, sup, None)

    def mk_extract(blk, buf, coloff, out_h):
        def extract(pos, rr):
            rrv = jnp.full((LANES,), rr + coloff, jnp.int32)
            bv = jnp.full((LANES,), buf, jnp.int32)
            slot = fire_row(pos)
            for k in range(EMB // LANES):
                d = k * LANES + iota
                v = plsc.load_gather(
                    blk, [bv, lax.shift_right_logical(d, 3), d & 7, rrv])
                stg[slot, pl.ds(k * LANES, LANES)] = v
            pltpu.async_copy(stg.at[slot], out_h.at[pl.ds(pos * EMB, EMB)],
                             outsem)
        return extract

    def mk_extract_tail(tail_v, out_h):
        def extract(pos, rr):
            rv_ = jnp.full((LANES,), rr, jnp.int32)
            slot = fire_row(pos)
            for k in range(EMB // LANES):
                v = plsc.load_gather(tail_v, [rv_, k * LANES + iota])
                stg[slot, pl.ds(k * LANES, LANES)] = v
            pltpu.async_copy(stg.at[slot], out_h.at[pl.ds(pos * EMB, EMB)],
                             outsem)
        return extract

    def fetch(p, buf):
        off = pl.multiple_of((jlo + 2 * p) * 128, 256)
        pltpu.async_copy(uet_h.at[:, :, pl.ds(off, 256)], ublk.at[buf], blksem)
        pltpu.async_copy(iet_h.at[:, :, pl.ds(off, 256)], iblk.at[buf], blksem)

    pcnt = lax.shift_right_logical(jcnt, 1)

    for p in range(DEPTH):
        fetch(p, p)

    def step(t, _):
        buf = t % DEPTH

        pltpu.make_async_copy(uet_h.at[:, :, pl.ds(0, 256)],
                              ublk.at[buf], blksem).wait()
        pltpu.make_async_copy(uet_h.at[:, :, pl.ds(0, 256)],
                              iblk.at[buf], blksem).wait()
        scan(ul, nu, 2 * t, mk_extract(ublk, buf, 0, uo_h))
        scan(il, ni, 2 * t, mk_extract(iblk, buf, 0, io_h))
        scan(ul, nu, 2 * t + 1, mk_extract(ublk, buf, 128, uo_h))
        scan(il, ni, 2 * t + 1, mk_extract(iblk, buf, 128, io_h))

        @pl.when(t + DEPTH < pcnt)
        def _():
            fetch(t + DEPTH, buf)

        return _

    lax.fori_loop(0, pcnt, step, None)

    scan(ul, nu, NJL - jlo, mk_extract_tail(tu_v, uo_h))
    scan(il, ni, NJL - jlo, mk_extract_tail(ti_v, io_h))

    def dr(i, _):
        pltpu.make_async_copy(uo_h.at[pl.ds(0, EMB)], drn, outsem).wait()
        return _

    lax.fori_loop(0, jnp.minimum(cnt_s[0], RING), dr, None)


def _dot_body(user_h, item_h, ubw_h, ibw_h, uo_h, io_h, out_h,
              u_idx, i_idx, ub_v, ib_v, ue_c, ie_c, out_v, sem):
    wid = lax.axis_index("s") * NC + lax.axis_index("c")
    base = wid * BPW
    iota = lax.iota(jnp.int32, LANES)

    for c in range(NCH):
        pltpu.sync_copy(user_h.at[pl.ds(base + c * CHUNK, CHUNK)], u_idx.at[c])
        pltpu.sync_copy(item_h.at[pl.ds(base + c * CHUNK, CHUNK)], i_idx.at[c])
    copies = []
    for c in range(NCH):
        sl = pl.ds(c * CHUNK, CHUNK)
        copies.append(pltpu.async_copy(ubw_h.at[u_idx.at[c]], ub_v.at[sl], sem))
        copies.append(pltpu.async_copy(ibw_h.at[i_idx.at[c]], ib_v.at[sl], sem))
    for cp in copies:
        cp.wait()

    def chunk_step(c, _):
        buf = c % 2
        roff = (base + c * CHUNK) * EMB
        cu = pltpu.async_copy(uo_h.at[pl.ds(roff, CHUNK * EMB)],
                              ue_c.at[buf], sem)
        ci = pltpu.async_copy(io_h.at[pl.ds(roff, CHUNK * EMB)],
                              ie_c.at[buf], sem)
        cu.wait()
        ci.wait()
        for g in range(CHUNK // LANES):
            gl = pl.ds(c * CHUNK + g * LANES, LANES)
            acc = ub_v[gl] + ib_v[gl]
            for l in range(LANES):
                e = (g * LANES + l) * EMB
                s = (ue_c[buf, pl.ds(e, LANES)] * ie_c[buf, pl.ds(e, LANES)])
                for k in range(1, EMB // LANES):
                    s = s + (ue_c[buf, pl.ds(e + k * LANES, LANES)]
                             * ie_c[buf, pl.ds(e + k * LANES, LANES)])
                dot = jnp.sum(s)
                acc = acc + jnp.where(iota == l, dot, 0.0)
            out_v[gl] = acc
        return _

    lax.fori_loop(0, NCH, chunk_step, None)
    pltpu.sync_copy(out_v, out_h.at[pl.ds(base, BPW)])


@jax.jit
def _mf_biased_sc(user, item, ubw, ibw, uew, iew):
    mesh = plsc.VectorSubcoreMesh(core_axis_name="c", subcore_axis_name="s")
    cp = pltpu.CompilerParams(needs_layout_passes=False,
                              use_tc_tiling_on_sc=True)
    uet = jnp.swapaxes(uew, 0, 1).reshape(8, 8, NROW)
    iet = jnp.swapaxes(iew, 0, 1).reshape(8, 8, NROW)
    tu = uew[TAIL0:]
    ti = iew[TAIL0:]
    ubw1 = ubw.reshape(-1)
    ibw1 = ibw.reshape(-1)

    ue_rows, ie_rows = pl.kernel(
        _extract_body,
        out_type=(jax.ShapeDtypeStruct((BATCH * EMB,), jnp.float32),
                  jax.ShapeDtypeStruct((BATCH * EMB,), jnp.float32)),
        mesh=mesh,
        compiler_params=cp,
        scratch_types=[
            pltpu.VMEM((ITMP,), jnp.int32),
            pltpu.VMEM((LCAP,), jnp.int32),
            pltpu.VMEM((LCAP,), jnp.int32),
            pltpu.VMEM((DEPTH, 8, 8, 256), jnp.float32),
            pltpu.VMEM((DEPTH, 8, 8, 256), jnp.float32),
            pltpu.VMEM((TAILN, EMB), jnp.float32),
            pltpu.VMEM((TAILN, EMB), jnp.float32),
            pltpu.VMEM((RING, EMB), jnp.float32),
            pltpu.VMEM((EMB,), jnp.float32),
            pltpu.SMEM((8,), jnp.int32),
            pltpu.SemaphoreType.DMA,
            pltpu.SemaphoreType.DMA,
        ],
    )(user, item, uet, iet, tu, ti)

    return pl.kernel(
        _dot_body,
        out_type=jax.ShapeDtypeStruct((BATCH,), jnp.float32),
        mesh=mesh,
        compiler_params=cp,
        scratch_types=[
            pltpu.VMEM((NCH, CHUNK), jnp.int32),
            pltpu.VMEM((NCH, CHUNK), jnp.int32),
            pltpu.VMEM((BPW,), jnp.float32),
            pltpu.VMEM((BPW,), jnp.float32),
            pltpu.VMEM((2, CHUNK * EMB), jnp.float32),
            pltpu.VMEM((2, CHUNK * EMB), jnp.float32),
            pltpu.VMEM((BPW,), jnp.float32),
            pltpu.SemaphoreType.DMA,
        ],
    )(user, item, ubw1, ibw1, ue_rows, ie_rows)


def kernel(user, item, user_biases_w, item_biases_w, user_emb_w, item_emb_w):
    return _mf_biased_sc(user, item, user_biases_w, item_biases_w,
                         user_emb_w, item_emb_w)


# speedup vs baseline: 1.3902x; 1.3902x over previous
import functools

import jax
import jax.numpy as jnp
from jax import lax
from jax.experimental import pallas as pl
from jax.experimental.pallas import tpu as pltpu
from jax.experimental.pallas import tpu_sc as plsc

BATCH = 16384
EMB = 64
NC = 2
NS = 16
LANES = 16
NW = NC * NS
BPW = BATCH // NW
CHUNK = 128
NCH = BPW // CHUNK
NROW = 1000000
NJ = (NROW + 127) // 128
NJL = NROW // 128
JPW = (NJ + NW - 1) // NW
TAIL0 = NJL * 128
TAILN = NROW - TAIL0
LCAP = BATCH + LANES
ITMP = 4096
RING = 16
DEPTH = 4
NBK = 544


def _scalar(x):
    return x if getattr(x, "ndim", 0) == 0 else x[0]


def _extract_body(user_h, item_h, uet_h, iet_h, tu_h, ti_h,
                  uo_h, io_h,
                  itmp, ul, il, ublk, iblk, tu_v, ti_v,
                  stg, drn, cnt_s, blksem, outsem):
    wid = lax.axis_index("s") * NC + lax.axis_index("c")
    jlo = wid * JPW
    jhi = jnp.minimum(jlo + JPW, NJ)
    jhb = jnp.minimum(jhi, NJL)
    jcnt = jhb - jlo
    iota = lax.iota(jnp.int32, LANES)

    pltpu.sync_copy(tu_h, tu_v)
    pltpu.sync_copy(ti_h, ti_v)

    def build(idx_h, lst):
        def piece(s, n):
            pltpu.sync_copy(idx_h.at[pl.ds(s * ITMP, ITMP)], itmp)

            def sup(c, n):
                for q in range(8):
                    rv = itmp[pl.ds(c * 128 + q * LANES, LANES)]
                    jv = lax.shift_right_logical(rv, 7)
                    m = (jv >= jlo) & (jv < jhi)
                    ent = (lax.shift_left(jv - jlo, 21)
                           | lax.shift_left(rv & 127, 14)
                           | (s * ITMP + c * 128 + q * LANES + iota))
                    plsc.store_compressed(lst.at[pl.ds(n, LANES)], ent,
                                          mask=m)
                    n = n + _scalar(plsc.all_reduce_population_count(m))
                return n

            return lax.fori_loop(0, ITMP // 128, sup, n)

        n = jnp.int32(0)
        for s in range(BATCH // ITMP):
            n = piece(s, n)
        return n

    nu = build(user_h, ul)
    ni = build(item_h, il)

    cnt_s[0] = 0

    def fire_row(pos):
        c = cnt_s[0]
        slot = c & (RING - 1)

        @pl.when(c >= RING)
        def _():
            pltpu.make_async_copy(uo_h.at[pl.ds(0, EMB)], drn, outsem).wait()

        cnt_s[0] = c + 1
        return slot

    def scan(lst, n, jrel, extract):
        nv = lax.shift_right_logical(n + 127, 7)

        def sup(c, _):
            evs, ms = [], []
            for q in range(8):
                base_e = c * 128 + q * LANES
                ev = lst[pl.ds(base_e, LANES)]
                m = ((lax.shift_right_logical(ev, 21) == jrel)
                     & ((base_e + iota) < n))
                evs.append(ev)
                ms.append(m)
            big = ms[0]
            for q in range(1, 8):
                big = big | ms[q]

            @pl.when(jnp.any(big))
            def _():
                for q in range(8):
                    base_e = c * 128 + q * LANES

                    @pl.when(jnp.any(ms[q]))
                    def _():
                        def w_cond(m_):
                            return jnp.any(m_)

                        def w_body(m_):
                            lane = _scalar(plsc.all_reduce_ffs(m_))
                            e = plsc.load_gather(
                                lst, [jnp.full((LANES,), base_e + lane,
                                               jnp.int32)])[0]
                            extract(e & 0x3FFF,
                                    lax.shift_right_logical(e, 14) & 127)
                            return m_ & (iota != lane)

                        lax.while_loop(w_cond, w_body, ms[q])

            return _

        lax.fori_loop(0, nv, sup, None)

    def mk_extract(blk, buf, out_h):
        def extract(pos, rr):
            rrv = jnp.full((LANES,), rr, jnp.int32)
            bv = jnp.full((LANES,), buf, jnp.int32)
            slot = fire_row(pos)
            for k in range(EMB // LANES):
                d = k * LANES + iota
                v = plsc.load_gather(
                    blk, [bv, lax.shift_right_logical(d, 3), d & 7, rrv])
                stg[slot, pl.ds(k * LANES, LANES)] = v
            pltpu.async_copy(stg.at[slot], out_h.at[pl.ds(pos * EMB, EMB)],
                             outsem)
        return extract

    def mk_extract_tail(tail_v, out_h):
        def extract(pos, rr):
            rv_ = jnp.full((LANES,), rr, jnp.int32)
            slot = fire_row(pos)
            for k in range(EMB // LANES):
                v = plsc.load_gather(tail_v, [rv_, k * LANES + iota])
                stg[slot, pl.ds(k * LANES, LANES)] = v
            pltpu.async_copy(stg.at[slot], out_h.at[pl.ds(pos * EMB, EMB)],
                             outsem)
        return extract

    def fetch(j, buf):
        off = pl.multiple_of(j * 128, 128)
        pltpu.async_copy(uet_h.at[:, :, pl.ds(off, 128)], ublk.at[buf], blksem)
        pltpu.async_copy(iet_h.at[:, :, pl.ds(off, 128)], iblk.at[buf], blksem)

    for p in range(DEPTH):
        fetch(jlo + p, p)

    def step(t, _):
        buf = t % DEPTH

        pltpu.make_async_copy(uet_h.at[:, :, pl.ds(0, 128)],
                              ublk.at[buf], blksem).wait()
        pltpu.make_async_copy(uet_h.at[:, :, pl.ds(0, 128)],
                              iblk.at[buf], blksem).wait()
        scan(ul, nu, t, mk_extract(ublk, buf, uo_h))
        scan(il, ni, t, mk_extract(iblk, buf, io_h))

        @pl.when(t + DEPTH < jcnt)
        def _():
            fetch(jlo + t + DEPTH, buf)

        return _

    lax.fori_loop(0, jcnt, step, None)

    scan(ul, nu, NJL - jlo, mk_extract_tail(tu_v, uo_h))
    scan(il, ni, NJL - jlo, mk_extract_tail(ti_v, io_h))

    def dr(i, _):
        pltpu.make_async_copy(uo_h.at[pl.ds(0, EMB)], drn, outsem).wait()
        return _

    lax.fori_loop(0, jnp.minimum(cnt_s[0], RING), dr, None)


def _dot_body(user_h, item_h, ubw_h, ibw_h, uo_h, io_h, out_h,
              u_idx, i_idx, ub_v, ib_v, ue_c, ie_c, out_v, sem):
    wid = lax.axis_index("s") * NC + lax.axis_index("c")
    base = wid * BPW
    iota = lax.iota(jnp.int32, LANES)

    for c in range(NCH):
        pltpu.sync_copy(user_h.at[pl.ds(base + c * CHUNK, CHUNK)], u_idx.at[c])
        pltpu.sync_copy(item_h.at[pl.ds(base + c * CHUNK, CHUNK)], i_idx.at[c])
    copies = []
    for c in range(NCH):
        sl = pl.ds(c * CHUNK, CHUNK)
        copies.append(pltpu.async_copy(ubw_h.at[u_idx.at[c]], ub_v.at[sl], sem))
        copies.append(pltpu.async_copy(ibw_h.at[i_idx.at[c]], ib_v.at[sl], sem))
    for cp in copies:
        cp.wait()

    def chunk_step(c, _):
        buf = c % 2
        roff = (base + c * CHUNK) * EMB
        cu = pltpu.async_copy(uo_h.at[pl.ds(roff, CHUNK * EMB)],
                              ue_c.at[buf], sem)
        ci = pltpu.async_copy(io_h.at[pl.ds(roff, CHUNK * EMB)],
                              ie_c.at[buf], sem)
        cu.wait()
        ci.wait()
        for g in range(CHUNK // LANES):
            gl = pl.ds(c * CHUNK + g * LANES, LANES)
            acc = ub_v[gl] + ib_v[gl]
            for l in range(LANES):
                e = (g * LANES + l) * EMB
                s = (ue_c[buf, pl.ds(e, LANES)] * ie_c[buf, pl.ds(e, LANES)])
                for k in range(1, EMB // LANES):
                    s = s + (ue_c[buf, pl.ds(e + k * LANES, LANES)]
                             * ie_c[buf, pl.ds(e + k * LANES, LANES)])
                dot = jnp.sum(s)
                acc = acc + jnp.where(iota == l, dot, 0.0)
            out_v[gl] = acc
        return _

    lax.fori_loop(0, NCH, chunk_step, None)
    pltpu.sync_copy(out_v, out_h.at[pl.ds(base, BPW)])


@jax.jit
def _mf_biased_sc(user, item, ubw, ibw, uew, iew):
    mesh = plsc.VectorSubcoreMesh(core_axis_name="c", subcore_axis_name="s")
    cp = pltpu.CompilerParams(needs_layout_passes=False,
                              use_tc_tiling_on_sc=True)
    uet = jnp.swapaxes(uew, 0, 1).reshape(8, 8, NROW)
    iet = jnp.swapaxes(iew, 0, 1).reshape(8, 8, NROW)
    tu = uew[TAIL0:]
    ti = iew[TAIL0:]
    ubw1 = ubw.reshape(-1)
    ibw1 = ibw.reshape(-1)

    ue_rows, ie_rows = pl.kernel(
        _extract_body,
        out_type=(jax.ShapeDtypeStruct((BATCH * EMB,), jnp.float32),
                  jax.ShapeDtypeStruct((BATCH * EMB,), jnp.float32)),
        mesh=mesh,
        compiler_params=cp,
        scratch_types=[
            pltpu.VMEM((ITMP,), jnp.int32),
            pltpu.VMEM((LCAP,), jnp.int32),
            pltpu.VMEM((LCAP,), jnp.int32),
            pltpu.VMEM((DEPTH, 8, 8, 128), jnp.float32),
            pltpu.VMEM((DEPTH, 8, 8, 128), jnp.float32),
            pltpu.VMEM((TAILN, EMB), jnp.float32),
            pltpu.VMEM((TAILN, EMB), jnp.float32),
            pltpu.VMEM((RING, EMB), jnp.float32),
            pltpu.VMEM((EMB,), jnp.float32),
            pltpu.SMEM((8,), jnp.int32),
            pltpu.SemaphoreType.DMA,
            pltpu.SemaphoreType.DMA,
        ],
    )(user, item, uet, iet, tu, ti)

    return pl.kernel(
        _dot_body,
        out_type=jax.ShapeDtypeStruct((BATCH,), jnp.float32),
        mesh=mesh,
        compiler_params=cp,
        scratch_types=[
            pltpu.VMEM((NCH, CHUNK), jnp.int32),
            pltpu.VMEM((NCH, CHUNK), jnp.int32),
            pltpu.VMEM((BPW,), jnp.float32),
            pltpu.VMEM((BPW,), jnp.float32),
            pltpu.VMEM((2, CHUNK * EMB), jnp.float32),
            pltpu.VMEM((2, CHUNK * EMB), jnp.float32),
            pltpu.VMEM((BPW,), jnp.float32),
            pltpu.SemaphoreType.DMA,
        ],
    )(user, item, ubw1, ibw1, ue_rows, ie_rows)


def kernel(user, item, user_biases_w, item_biases_w, user_emb_w, item_emb_w):
    return _mf_biased_sc(user, item, user_biases_w, item_biases_w,
                         user_emb_w, item_emb_w)
